# baseline (speedup 1.0000x reference)
import functools

import jax
import jax.numpy as jnp
from jax import lax
from jax.experimental import pallas as pl
from jax.experimental.pallas import tpu as pltpu
from jax.experimental.pallas import tpu_sc as plsc

_NC = 2
_NS = 16
_NW = _NC * _NS
_CHUNK = 64
_NBUF = 4
_AHEAD = 2
_STAGE = 40


def _make_agg_kernel(n_acc: int, d: int, n_chunks: int):
    rows_per_tile = n_acc // _NS
    n_stages = n_chunks // _STAGE
    mesh = plsc.VectorSubcoreMesh(core_axis_name="c", subcore_axis_name="s")

    @functools.partial(
        pl.kernel,
        out_type=jax.ShapeDtypeStruct((_NC, n_acc, d), jnp.float32),
        mesh=mesh,
        scratch_types=[
            pltpu.VMEM((_STAGE, _CHUNK), jnp.int32),
            pltpu.VMEM((_STAGE, _CHUNK), jnp.int32),
            pltpu.VMEM((_STAGE, _CHUNK), jnp.float32),
            pltpu.VMEM((_NBUF, _CHUNK, d), jnp.float32),
            pltpu.VMEM_SHARED((n_acc, d), jnp.float32),
            pltpu.SemaphoreType.DMA,
            pltpu.SemaphoreType.DMA,
        ],
    )
    def agg(x_hbm, src_hbm, dst_hbm, w_hbm, out_hbm,
            src_v, dst_v, w_v, bufs, accum, gsem, ssem):
        cid = lax.axis_index("c")
        sid = lax.axis_index("s")
        wid = cid * _NS + sid

        zvec = jnp.zeros((16,), jnp.float32)

        def zrow(r, carry):
            for q in range(d // 16):
                bufs[0, r, pl.ds(q * 16, 16)] = zvec
            return carry

        lax.fori_loop(0, _CHUNK, zrow, 0, unroll=False)
        row0 = sid * rows_per_tile
        full, rem = divmod(rows_per_tile, _CHUNK)
        for k in range(full):
            pltpu.sync_copy(bufs.at[0],
                            accum.at[pl.ds(row0 + k * _CHUNK, _CHUNK)])
        if rem:
            pltpu.sync_copy(bufs.at[0, pl.ds(0, rem)],
                            accum.at[pl.ds(row0 + full * _CHUNK, rem)])
        plsc.subcore_barrier()

        def run_stage(h, carry):
            blk = wid * n_stages + h
            pltpu.sync_copy(src_hbm.at[blk], src_v)
            pltpu.sync_copy(dst_hbm.at[blk], dst_v)
            pltpu.sync_copy(w_hbm.at[blk], w_v)

            half = _CHUNK // 2

            def issue_gather(c, b):
                for q in range(2):
                    qs = pl.ds(q * half, half)
                    pltpu.async_copy(x_hbm.at[src_v.at[c, qs]],
                                     bufs.at[b, qs], gsem)

            def wait_gather(c, b):
                for q in range(2):
                    qs = pl.ds(q * half, half)
                    pltpu.make_async_copy(x_hbm.at[src_v.at[c, qs]],
                                          bufs.at[b, qs], gsem).wait()

            for p in range(_AHEAD):
                issue_gather(p, p)

            def ring_body(cc, carry2):
                for bi in range(_NBUF):
                    c = cc * _NBUF + bi
                    wait_gather(c, bi)
                    nb = (bi + _AHEAD) % _NBUF

                    @pl.when(c >= _NBUF - _AHEAD)
                    def _():
                        pltpu.make_async_copy(
                            bufs.at[nb],
                            accum.at[dst_v.at[c - (_NBUF - _AHEAD)]],
                            ssem).wait()

                    @pl.when(c + _AHEAD < _STAGE)
                    def _():
                        issue_gather(c + _AHEAD, nb)

                    @plsc.parallel_loop(0, _CHUNK // 16)
                    def group_body(g):
                        wvec = w_v[c, pl.ds(g * 16, 16)]
                        for e in range(16):
                            wgt = wvec[e]
                            row = g * 16 + e
                            for r in range(d // 16):
                                sl = pl.ds(r * 16, 16)
                                bufs[bi, row, sl] = bufs[bi, row, sl] * wgt

                    pltpu.async_copy(
                        bufs.at[bi], accum.at[dst_v.at[c]], ssem, add=True)
                return carry2

            lax.fori_loop(0, _STAGE // _NBUF, ring_body, 0, unroll=False)

            for c in range(_STAGE - (_NBUF - _AHEAD), _STAGE):
                pltpu.make_async_copy(
                    bufs.at[c % _NBUF], accum.at[dst_v.at[c]], ssem).wait()
            return carry

        lax.fori_loop(0, n_stages, run_stage, 0, unroll=False)
        plsc.subcore_barrier()

        pltpu.sync_copy(accum.at[pl.ds(row0, rows_per_tile)],
                        out_hbm.at[cid, pl.ds(row0, rows_per_tile)])

    return agg


def _mm_body(p0_ref, p1_ref, w_ref, b_ref, o_ref):
    z = p0_ref[0] + p1_ref[0]
    o_ref[...] = (
        jnp.dot(z, w_ref[...], preferred_element_type=jnp.float32,
                precision=lax.Precision.HIGHEST)
        + b_ref[...]
    )


def kernel(x, edge_index, edge_weight, W, b):
    n_nodes, d_in = x.shape
    d_out = W.shape[1]
    n_edges = edge_index.shape[1]

    epw = -(-n_edges // (_NW * _STAGE * _CHUNK)) * (_STAGE * _CHUNK)
    n_chunks = epw // _CHUNK
    pad = epw * _NW - n_edges

    src = edge_index[0].astype(jnp.int32)
    dst = edge_index[1].astype(jnp.int32)
    w = edge_weight.astype(jnp.float32)
    if pad:
        pad_idx = jnp.arange(pad, dtype=jnp.int32) % n_nodes
        src = jnp.concatenate([src, pad_idx])
        dst = jnp.concatenate([dst, pad_idx])
        w = jnp.concatenate([w, jnp.zeros((pad,), jnp.float32)])
    n_stages = n_chunks // _STAGE
    src = src.reshape(_NW * n_stages, _STAGE, _CHUNK)
    dst = dst.reshape(_NW * n_stages, _STAGE, _CHUNK)
    w = w.reshape(_NW * n_stages, _STAGE, _CHUNK)

    n_acc = -(-n_nodes // (8 * _NS)) * (8 * _NS)

    partial = _make_agg_kernel(n_acc, d_in, n_chunks)(x, src, dst, w)

    blk = 1000
    grid = n_nodes // blk
    out = pl.pallas_call(
        _mm_body,
        grid=(grid,),
        in_specs=[
            pl.BlockSpec((1, blk, d_in), lambda i: (0, i, 0)),
            pl.BlockSpec((1, blk, d_in), lambda i: (1, i, 0)),
            pl.BlockSpec((d_in, d_out), lambda i: (0, 0)),
            pl.BlockSpec((1, d_out), lambda i: (0, 0)),
        ],
        out_specs=pl.BlockSpec((blk, d_out), lambda i: (i, 0)),
        out_shape=jax.ShapeDtypeStruct((n_nodes, d_out), jnp.float32),
    )(partial, partial, W, b.reshape(1, d_out))
    return out


# speedup vs baseline: 1.0659x; 1.0659x over previous
import functools

import jax
import jax.numpy as jnp
from jax import lax
from jax.experimental import pallas as pl
from jax.experimental.pallas import tpu as pltpu
from jax.experimental.pallas import tpu_sc as plsc

_NC = 2
_NS = 16
_NW = _NC * _NS
_CHUNK = 64
_NBUF = 4
_AHEAD = 2
_STAGE = 40


def _make_agg_kernel(n_acc: int, d: int, n_chunks: int):
    rows_per_tile = n_acc // _NS
    n_stages = n_chunks // _STAGE
    mesh = plsc.VectorSubcoreMesh(core_axis_name="c", subcore_axis_name="s")

    @functools.partial(
        pl.kernel,
        out_type=jax.ShapeDtypeStruct((_NC, n_acc, d), jnp.float32),
        mesh=mesh,
        scratch_types=[
            pltpu.VMEM((_STAGE, _CHUNK), jnp.int32),
            pltpu.VMEM((_STAGE, _CHUNK), jnp.int32),
            pltpu.VMEM((_STAGE, _CHUNK), jnp.float32),
            pltpu.VMEM((_NBUF, _CHUNK, d), jnp.float32),
            pltpu.VMEM_SHARED((n_acc, d), jnp.float32),
            pltpu.SemaphoreType.DMA,
            pltpu.SemaphoreType.DMA,
        ],
    )
    def agg(x_hbm, src_hbm, dst_hbm, w_hbm, out_hbm,
            src_v, dst_v, w_v, bufs, accum, gsem, ssem):
        cid = lax.axis_index("c")
        sid = lax.axis_index("s")
        wid = cid * _NS + sid

        zvec = jnp.zeros((16,), jnp.float32)

        def zrow(r, carry):
            for q in range(d // 16):
                bufs[0, r, pl.ds(q * 16, 16)] = zvec
            return carry

        lax.fori_loop(0, _CHUNK, zrow, 0, unroll=False)
        row0 = sid * rows_per_tile
        full, rem = divmod(rows_per_tile, _CHUNK)
        for k in range(full):
            pltpu.sync_copy(bufs.at[0],
                            accum.at[pl.ds(row0 + k * _CHUNK, _CHUNK)])
        if rem:
            pltpu.sync_copy(bufs.at[0, pl.ds(0, rem)],
                            accum.at[pl.ds(row0 + full * _CHUNK, rem)])
        plsc.subcore_barrier()

        def run_stage(h, carry):
            blk = wid * n_stages + h
            pltpu.sync_copy(src_hbm.at[blk], src_v)
            pltpu.sync_copy(dst_hbm.at[blk], dst_v)
            pltpu.sync_copy(w_hbm.at[blk], w_v)

            for p in range(_AHEAD):
                pltpu.async_copy(x_hbm.at[src_v.at[p]], bufs.at[p], gsem)

            def ring_body(cc, carry2):
                for bi in range(_NBUF):
                    c = cc * _NBUF + bi
                    pltpu.make_async_copy(
                        x_hbm.at[src_v.at[c]], bufs.at[bi], gsem).wait()
                    nb = (bi + _AHEAD) % _NBUF

                    @pl.when(c >= _NBUF - _AHEAD)
                    def _():
                        pltpu.make_async_copy(
                            bufs.at[nb],
                            accum.at[dst_v.at[c - (_NBUF - _AHEAD)]],
                            ssem).wait()

                    @pl.when(c + _AHEAD < _STAGE)
                    def _():
                        pltpu.async_copy(
                            x_hbm.at[src_v.at[c + _AHEAD]], bufs.at[nb],
                            gsem)

                    def group_body(g, carry3):
                        wvec = w_v[c, pl.ds(g * 16, 16)]
                        for e in range(16):
                            wgt = wvec[e]
                            row = g * 16 + e
                            for r in range(d // 16):
                                sl = pl.ds(r * 16, 16)
                                bufs[bi, row, sl] = bufs[bi, row, sl] * wgt
                        return carry3

                    lax.fori_loop(0, _CHUNK // 16, group_body, 0,
                                  unroll=False)

                    pltpu.async_copy(
                        bufs.at[bi], accum.at[dst_v.at[c]], ssem, add=True)
                return carry2

            lax.fori_loop(0, _STAGE // _NBUF, ring_body, 0, unroll=False)

            for c in range(_STAGE - (_NBUF - _AHEAD), _STAGE):
                pltpu.make_async_copy(
                    bufs.at[c % _NBUF], accum.at[dst_v.at[c]], ssem).wait()
            return carry

        lax.fori_loop(0, n_stages, run_stage, 0, unroll=False)
        plsc.subcore_barrier()

        pltpu.sync_copy(accum.at[pl.ds(row0, rows_per_tile)],
                        out_hbm.at[cid, pl.ds(row0, rows_per_tile)])

    return agg


def _mm_body(p0_ref, p1_ref, w_ref, b_ref, o_ref):
    z = p0_ref[0] + p1_ref[0]
    o_ref[...] = (
        jnp.dot(z, w_ref[...], preferred_element_type=jnp.float32,
                precision=lax.Precision.HIGHEST)
        + b_ref[...]
    )


def kernel(x, edge_index, edge_weight, W, b):
    n_nodes, d_in = x.shape
    d_out = W.shape[1]
    n_edges = edge_index.shape[1]

    epw = -(-n_edges // (_NW * _STAGE * _CHUNK)) * (_STAGE * _CHUNK)
    n_chunks = epw // _CHUNK
    pad = epw * _NW - n_edges

    src = edge_index[0].astype(jnp.int32)
    dst = edge_index[1].astype(jnp.int32)
    w = edge_weight.astype(jnp.float32)
    if pad:
        pad_idx = jnp.arange(pad, dtype=jnp.int32) % n_nodes
        src = jnp.concatenate([src, pad_idx])
        dst = jnp.concatenate([dst, pad_idx])
        w = jnp.concatenate([w, jnp.zeros((pad,), jnp.float32)])
    n_stages = n_chunks // _STAGE
    src = src.reshape(_NW * n_stages, _STAGE, _CHUNK)
    dst = dst.reshape(_NW * n_stages, _STAGE, _CHUNK)
    w = w.reshape(_NW * n_stages, _STAGE, _CHUNK)

    n_acc = -(-n_nodes // (8 * _NS)) * (8 * _NS)

    partial = _make_agg_kernel(n_acc, d_in, n_chunks)(x, src, dst, w)

    blk = 1000
    grid = n_nodes // blk
    out = pl.pallas_call(
        _mm_body,
        grid=(grid,),
        in_specs=[
            pl.BlockSpec((1, blk, d_in), lambda i: (0, i, 0)),
            pl.BlockSpec((1, blk, d_in), lambda i: (1, i, 0)),
            pl.BlockSpec((d_in, d_out), lambda i: (0, 0)),
            pl.BlockSpec((1, d_out), lambda i: (0, 0)),
        ],
        out_specs=pl.BlockSpec((blk, d_out), lambda i: (i, 0)),
        out_shape=jax.ShapeDtypeStruct((n_nodes, d_out), jnp.float32),
    )(partial, partial, W, b.reshape(1, d_out))
    return out
